# baseline (speedup 1.0000x reference)
import dataclasses

import jax
import jax.numpy as jnp
from jax import lax
from jax.experimental import pallas as pl
from jax.experimental.pallas import tpu as pltpu
from jax.experimental.pallas import tpu_sc as plsc

DIM = 64
BATCH = 16384
REG_COEF = 1e-05
NC = 2
NS = 16
LANES = 16
NW = NC * NS
BPW = BATCH // NW
CHUNK = 128
NCHUNK = BPW // CHUNK
GROUPS = BPW // LANES


def _sc_body(uidx_hbm, pidx_hbm, nidx_hbm, utab_hbm, itab_hbm,
             diff_hbm, sq_hbm,
             idx_u, idx_p, idx_n, rows_u, rows_p, rows_n,
             scores_v, sq_v, sem):
    wid = lax.axis_index("s") * NC + lax.axis_index("c")

    pltpu.sync_copy(uidx_hbm.at[wid], idx_u)
    pltpu.sync_copy(pidx_hbm.at[wid], idx_p)
    pltpu.sync_copy(nidx_hbm.at[wid], idx_n)

    copies = []
    for j in range(NCHUNK):
        dst = pl.ds(j * CHUNK, CHUNK)
        copies.append(pltpu.async_copy(utab_hbm.at[idx_u.at[j]], rows_u.at[dst], sem))
        copies.append(pltpu.async_copy(itab_hbm.at[idx_p.at[j]], rows_p.at[dst], sem))
        copies.append(pltpu.async_copy(itab_hbm.at[idx_n.at[j]], rows_n.at[dst], sem))
    for c in copies:
        c.wait()

    sq_v[...] = jnp.zeros((LANES,), jnp.float32)
    iota = lax.iota(jnp.int32, LANES)

    @pl.loop(0, GROUPS)
    def _group(g):
        row = g * LANES + iota
        pos = jnp.zeros((LANES,), jnp.float32)
        neg = jnp.zeros((LANES,), jnp.float32)
        sq = jnp.zeros((LANES,), jnp.float32)
        for d in range(DIM):
            col = jnp.full((LANES,), d, jnp.int32)
            u = plsc.load_gather(rows_u, [row, col])
            p = plsc.load_gather(rows_p, [row, col])
            n = plsc.load_gather(rows_n, [row, col])
            pos = pos + u * p
            neg = neg + u * n
            sq = sq + (u * u + p * p + n * n)
        scores_v[pl.ds(g * LANES, LANES)] = pos - neg
        sq_v[...] += sq

    pltpu.sync_copy(scores_v, diff_hbm.at[pl.ds(wid * BPW, BPW)])
    pltpu.sync_copy(sq_v, sq_hbm.at[wid])


def _loss_body(diff_ref, sq_ref, out_ref):
    d = diff_ref[...]
    sp = jnp.maximum(-d, 0.0) + jnp.log1p(jnp.exp(-jnp.abs(d)))
    out_ref[0] = jnp.sum(sp) * (1.0 / BATCH)
    out_ref[1] = (0.5 * REG_COEF) * jnp.sum(sq_ref[...])


@jax.jit
def kernel(userids, itemids_pos, itemids_neg, user_table, item_table):
    uidx = userids.astype(jnp.int32).reshape(NW, NCHUNK, CHUNK)
    pidx = itemids_pos.astype(jnp.int32).reshape(NW, NCHUNK, CHUNK)
    nidx = itemids_neg.astype(jnp.int32).reshape(NW, NCHUNK, CHUNK)

    mesh = plsc.VectorSubcoreMesh(
        core_axis_name="c", subcore_axis_name="s",
        num_cores=NC, num_subcores=NS)

    cp = pltpu.CompilerParams()
    if "needs_layout_passes" in pltpu.CompilerParams.__dataclass_fields__:
        cp = dataclasses.replace(cp, needs_layout_passes=False)
    if "use_tc_tiling_on_sc" in pltpu.CompilerParams.__dataclass_fields__:
        cp = dataclasses.replace(cp, use_tc_tiling_on_sc=False)

    sc = pl.kernel(
        _sc_body,
        compiler_params=cp,
        out_type=[
            jax.ShapeDtypeStruct((BATCH,), jnp.float32),
            jax.ShapeDtypeStruct((NW, LANES), jnp.float32),
        ],
        mesh=mesh,
        scratch_types=[
            pltpu.VMEM((NCHUNK, CHUNK), jnp.int32),
            pltpu.VMEM((NCHUNK, CHUNK), jnp.int32),
            pltpu.VMEM((NCHUNK, CHUNK), jnp.int32),
            pltpu.VMEM((BPW, DIM), jnp.float32),
            pltpu.VMEM((BPW, DIM), jnp.float32),
            pltpu.VMEM((BPW, DIM), jnp.float32),
            pltpu.VMEM((BPW,), jnp.float32),
            pltpu.VMEM((LANES,), jnp.float32),
            pltpu.SemaphoreType.DMA,
        ],
    )
    diff, sq = sc(uidx, pidx, nidx, user_table, item_table)

    out = pl.pallas_call(
        _loss_body,
        out_shape=jax.ShapeDtypeStruct((2,), jnp.float32),
        out_specs=pl.BlockSpec(memory_space=pltpu.SMEM),
    )(diff.reshape(BATCH // 128, 128), sq)
    return out[0], out[1]
